# baseline (speedup 1.0000x reference)
import functools

import jax
import jax.numpy as jnp
from jax import lax
from jax.experimental import pallas as pl
from jax.experimental.pallas import tpu as pltpu
from jax.experimental.pallas import tpu_sc as plsc

C = 21
NPIX = 4 * 512 * 512
NB = 4
PLANE = 512 * 512
NW = 32
CHUNK = PLANE // NW
NV = CHUNK // 16
B = 256
HB = 2 * B
LSTRIDE = HB + 1
UNROLL = 8
MAGIC = float(2 ** 23)
MAGIC_I = 0x4B000000


def _sc_hist_body(inp_hbm, tgt_hbm, out_hbm, tgt_v, inbuf_v, hist_v, flush_v,
                  sem0, sem1):
    cid = lax.axis_index("c")
    sid = lax.axis_index("s")
    wid = sid * 2 + cid
    base = wid * CHUNK

    ones16 = jnp.ones((16,), jnp.float32)
    zero16 = jnp.zeros((16,), jnp.float32)
    sems = (sem0, sem1)

    scale = float(B - 1)
    lanef = lax.iota(jnp.int32, 16).astype(jnp.float32) * float(LSTRIDE)
    hsz = float(16 * LSTRIDE)
    sel_pos = lanef + float(B) + scale + MAGIC
    sel_neg = lanef + MAGIC
    sel_pos2 = sel_pos + hsz
    sel_neg2 = sel_neg + hsz

    row0 = wid * 16

    pltpu.async_copy(inp_hbm.at[0, pl.ds(row0, 16), :], inbuf_v.at[0], sem0)
    pltpu.async_copy(inp_hbm.at[1, pl.ds(row0, 16), :], inbuf_v.at[1], sem0)

    tdescs = [pltpu.async_copy(tgt_hbm.at[b, pl.ds(row0, 16), :],
                               tgt_v.at[b], sem1)
              for b in range(NB)]
    for d in tdescs:
        d.wait()

    @plsc.parallel_loop(0, (2 * 16 * LSTRIDE + 15) // 16, 1, unroll=4)
    def zbody(i):
        hist_v[pl.ds(i * 16, 16)] = zero16

    def flush_class(roff, out_slice):
        @plsc.parallel_loop(0, HB // 16, 1, unroll=2)
        def fbody(kk):
            vals = []
            for l in range(16):
                off = roff + l * LSTRIDE + kk * 16
                vals.append(hist_v[pl.ds(off, 16)])
                hist_v[pl.ds(off, 16)] = zero16
            while len(vals) > 1:
                vals = [a + bb for a, bb in zip(vals[::2], vals[1::2])]
            flush_v[pl.ds(kk * 16, 16)] = vals[0]
        pltpu.sync_copy(flush_v, out_slice)

    def pair_body(k, _):
        c1 = 2 * k
        for b in range(NB):
            sslot = (b % 2) * 2
            nslot = ((b + 1) % 2) * 2
            if b < NB - 1:
                nbase = (b + 1) * C + c1
            else:
                nbase = jnp.minimum(c1 + 2, C - 3)
            pltpu.async_copy(inp_hbm.at[nbase, pl.ds(row0, 16), :],
                             inbuf_v.at[nslot], sems[(b + 1) % 2])
            pltpu.async_copy(inp_hbm.at[nbase + 1, pl.ds(row0, 16), :],
                             inbuf_v.at[nslot + 1], sems[(b + 1) % 2])
            pltpu.make_async_copy(inp_hbm.at[b * C + c1, pl.ds(row0, 16), :],
                                  inbuf_v.at[sslot], sems[b % 2]).wait()
            pltpu.make_async_copy(inp_hbm.at[b * C + c1 + 1,
                                             pl.ds(row0, 16), :],
                                  inbuf_v.at[sslot + 1], sems[b % 2]).wait()

            @plsc.parallel_loop(0, NV, 1, unroll=UNROLL)
            def iter_body(i):
                r = i >> 5
                jj = (i & 31) * 16
                t = tgt_v[b, r, pl.ds(jj, 16)]
                p1 = inbuf_v[sslot, r, pl.ds(jj, 16)]
                p2 = inbuf_v[sslot + 1, r, pl.ds(jj, 16)]
                is1 = t == c1
                comb1 = (jnp.where(is1, sel_pos, sel_neg)
                         + p1 * jnp.where(is1, -scale, scale))
                idx1 = plsc.bitcast(comb1, jnp.int32) - MAGIC_I
                plsc.addupdate_scatter(hist_v, [idx1], ones16)
                is2 = t == c1 + 1
                comb2 = (jnp.where(is2, sel_pos2, sel_neg2)
                         + p2 * jnp.where(is2, -scale, scale))
                idx2 = plsc.bitcast(comb2, jnp.int32) - MAGIC_I
                plsc.addupdate_scatter(hist_v, [idx2], ones16)

        flush_class(0, out_hbm.at[wid * C + c1])
        flush_class(16 * LSTRIDE, out_hbm.at[wid * C + c1 + 1])
        return 0
    lax.fori_loop(0, C // 2, pair_body, 0)

    pltpu.make_async_copy(inp_hbm.at[C - 3, pl.ds(row0, 16), :],
                          inbuf_v.at[0], sem0).wait()
    pltpu.make_async_copy(inp_hbm.at[C - 2, pl.ds(row0, 16), :],
                          inbuf_v.at[1], sem0).wait()

    cl = C - 1
    for b in range(NB):
        slot = b % 2
        pltpu.sync_copy(inp_hbm.at[b * C + cl, pl.ds(row0, 16), :],
                        inbuf_v.at[slot])

        @plsc.parallel_loop(0, NV, 1, unroll=UNROLL)
        def last_body(i):
            r = i >> 5
            jj = (i & 31) * 16
            t = tgt_v[b, r, pl.ds(jj, 16)]
            p = inbuf_v[slot, r, pl.ds(jj, 16)]
            isc = t == cl
            comb = (jnp.where(isc, sel_pos, sel_neg)
                    + p * jnp.where(isc, -scale, scale))
            idx = plsc.bitcast(comb, jnp.int32) - MAGIC_I
            plsc.addupdate_scatter(hist_v, [idx], ones16)

    flush_class(0, out_hbm.at[wid * C + cl])


def _sc_hist(inp3, tgt2):
    mesh = plsc.VectorSubcoreMesh(core_axis_name="c", subcore_axis_name="s",
                                  num_cores=2, num_subcores=16)
    return pl.kernel(
        _sc_hist_body,
        out_type=jax.ShapeDtypeStruct((NW * C, HB), jnp.float32),
        mesh=mesh,
        compiler_params=pltpu.CompilerParams(needs_layout_passes=False),
        scratch_types=[
            pltpu.VMEM((NB, 16, 512), jnp.int32),
            pltpu.VMEM((4, 16, 512), jnp.float32),
            pltpu.VMEM((2 * 16 * LSTRIDE,), jnp.float32),
            pltpu.VMEM((HB,), jnp.float32),
            pltpu.SemaphoreType.DMA,
            pltpu.SemaphoreType.DMA,
        ],
    )(inp3, tgt2)


def _tc_finish_body(hist_ref, out_ref):
    hv = hist_ref[...]
    sel = (lax.broadcasted_iota(jnp.int32, (C, NW * C), 1) % C
           == lax.broadcasted_iota(jnp.int32, (C, NW * C), 0)
           ).astype(jnp.float32)
    h = jnp.dot(sel, hv, preferred_element_type=jnp.float32)
    n0 = h[:, :B]
    n1 = h[:, B:]
    n = n0 + n1
    g = jnp.sum(n1, axis=1, keepdims=True)

    tri = (lax.broadcasted_iota(jnp.int32, (B, B), 0)
           <= lax.broadcasted_iota(jnp.int32, (B, B), 1)).astype(jnp.float32)
    cn = jnp.dot(n, tri, preferred_element_type=jnp.float32)
    cn1 = jnp.dot(n1, tri, preferred_element_type=jnp.float32)

    midn = (float(NPIX) - cn) + 0.5 * n
    midn1 = (g - cn1) + 0.5 * n1
    inter = g - midn1
    union = g + midn - midn1
    j = jnp.where(union > 0.0, 1.0 - inter / jnp.where(union > 0.0, union, 1.0),
                  0.0)
    s = float(B - 1)
    kk = lax.broadcasted_iota(jnp.int32, (1, B), 1)
    w = jnp.where((kk == 0) | (kk == B - 1), 0.5 / s, 1.0 / s)
    out_ref[...] = (jnp.sum(j * w) / float(C)).reshape(1, 1)


def _tc_finish(hist):
    return pl.pallas_call(
        _tc_finish_body,
        out_shape=jax.ShapeDtypeStruct((1, 1), jnp.float32),
    )(hist)


@jax.jit
def kernel(inputs, targets):
    inp3 = inputs.reshape(NB * C, 512, 512)
    hist = _sc_hist(inp3, targets)
    return _tc_finish(hist).reshape(())


# speedup vs baseline: 1.0139x; 1.0139x over previous
import functools

import jax
import jax.numpy as jnp
from jax import lax
from jax.experimental import pallas as pl
from jax.experimental.pallas import tpu as pltpu
from jax.experimental.pallas import tpu_sc as plsc

C = 21
NPIX = 4 * 512 * 512
NB = 4
PLANE = 512 * 512
NW = 32
CHUNK = PLANE // NW
NV = CHUNK // 16
B = 256
HB = 2 * B
LSTRIDE = HB + 1
UNROLL = 8
MAGIC = float(2 ** 23)
MAGIC_I = 0x4B000000


def _sc_hist_body(inp_hbm, tgt_hbm, out_hbm, tgt_v, inbuf_v, hist_v, flush_v,
                  sem0, sem1):
    cid = lax.axis_index("c")
    sid = lax.axis_index("s")
    wid = sid * 2 + cid
    base = wid * CHUNK

    ones16 = jnp.ones((16,), jnp.float32)
    zero16 = jnp.zeros((16,), jnp.float32)
    sems = (sem0, sem1)

    scale = float(B - 1)
    lanef = lax.iota(jnp.int32, 16).astype(jnp.float32) * float(LSTRIDE)
    sel_pos = lanef + float(B) + scale + MAGIC
    sel_neg = lanef + MAGIC

    row0 = wid * 16

    pltpu.async_copy(inp_hbm.at[0, pl.ds(row0, 16), :], inbuf_v.at[0], sem0)

    tdescs = [pltpu.async_copy(tgt_hbm.at[b, pl.ds(row0, 16), :],
                               tgt_v.at[b], sem1)
              for b in range(NB)]
    for d in tdescs:
        d.wait()

    @plsc.parallel_loop(0, (16 * LSTRIDE + 15) // 16, 1, unroll=4)
    def zbody(i):
        hist_v[pl.ds(i * 16, 16)] = zero16

    def class_body(c, _):
        for b in range(NB):
            slot = b % 2
            nslot = (b + 1) % 2
            if b < NB - 1:
                pltpu.async_copy(
                    inp_hbm.at[(b + 1) * C + c, pl.ds(row0, 16), :],
                    inbuf_v.at[nslot], sems[nslot])
            else:
                nc = jnp.minimum(c + 1, C - 1)
                pltpu.async_copy(
                    inp_hbm.at[nc, pl.ds(row0, 16), :],
                    inbuf_v.at[nslot], sems[nslot])
            pltpu.make_async_copy(
                inp_hbm.at[b * C + c, pl.ds(row0, 16), :],
                inbuf_v.at[slot], sems[slot]).wait()

            @plsc.parallel_loop(0, NV, 1, unroll=UNROLL)
            def iter_body(i):
                r = i >> 5
                jj = (i & 31) * 16
                p = inbuf_v[slot, r, pl.ds(jj, 16)]
                t = tgt_v[b, r, pl.ds(jj, 16)]
                isc = t == c
                comb = (jnp.where(isc, sel_pos, sel_neg)
                        + p * jnp.where(isc, -scale, scale))
                idx = plsc.bitcast(comb, jnp.int32) - MAGIC_I
                plsc.addupdate_scatter(hist_v, [idx], ones16)

        @plsc.parallel_loop(0, HB // 16, 1, unroll=2)
        def fbody(kk):
            vals = []
            for l in range(16):
                off = l * LSTRIDE + kk * 16
                vals.append(hist_v[pl.ds(off, 16)])
                hist_v[pl.ds(off, 16)] = zero16
            while len(vals) > 1:
                vals = [a + bb for a, bb in zip(vals[::2], vals[1::2])]
            flush_v[pl.ds(kk * 16, 16)] = vals[0]

        pltpu.sync_copy(flush_v, out_hbm.at[wid, c])
        return 0
    lax.fori_loop(0, C, class_body, 0)

    pltpu.make_async_copy(inp_hbm.at[C - 1, pl.ds(row0, 16), :],
                          inbuf_v.at[0], sem0).wait()


def _sc_hist(inp3, tgt2):
    mesh = plsc.VectorSubcoreMesh(core_axis_name="c", subcore_axis_name="s",
                                  num_cores=2, num_subcores=16)
    return pl.kernel(
        _sc_hist_body,
        out_type=jax.ShapeDtypeStruct((NW, C, HB), jnp.float32),
        mesh=mesh,
        compiler_params=pltpu.CompilerParams(needs_layout_passes=False),
        scratch_types=[
            pltpu.VMEM((NB, 16, 512), jnp.int32),
            pltpu.VMEM((2, 16, 512), jnp.float32),
            pltpu.VMEM((16 * LSTRIDE,), jnp.float32),
            pltpu.VMEM((HB,), jnp.float32),
            pltpu.SemaphoreType.DMA,
            pltpu.SemaphoreType.DMA,
        ],
    )(inp3, tgt2)


def _tc_finish_body(hist_ref, out_ref):
    h = jnp.sum(hist_ref[...], axis=0)
    n0 = h[:, :B]
    n1 = h[:, B:]
    n = n0 + n1
    g = jnp.sum(n1, axis=1, keepdims=True)

    tri = (lax.broadcasted_iota(jnp.int32, (B, B), 0)
           <= lax.broadcasted_iota(jnp.int32, (B, B), 1)).astype(jnp.float32)
    cn = jnp.dot(n, tri, preferred_element_type=jnp.float32)
    cn1 = jnp.dot(n1, tri, preferred_element_type=jnp.float32)

    midn = (float(NPIX) - cn) + 0.5 * n
    midn1 = (g - cn1) + 0.5 * n1
    inter = g - midn1
    union = g + midn - midn1
    j = jnp.where(union > 0.0, 1.0 - inter / jnp.where(union > 0.0, union, 1.0),
                  0.0)
    s = float(B - 1)
    kk = lax.broadcasted_iota(jnp.int32, (1, B), 1)
    w = jnp.where((kk == 0) | (kk == B - 1), 0.5 / s, 1.0 / s)
    out_ref[...] = (jnp.sum(j * w) / float(C)).reshape(1, 1)


def _tc_finish(hist):
    return pl.pallas_call(
        _tc_finish_body,
        out_shape=jax.ShapeDtypeStruct((1, 1), jnp.float32),
    )(hist)


@jax.jit
def kernel(inputs, targets):
    inp3 = inputs.reshape(NB * C, 512, 512)
    hist = _sc_hist(inp3, targets)
    return _tc_finish(hist).reshape(())
